# baseline (speedup 1.0000x reference)
import jax
import jax.numpy as jnp
from jax.experimental import pallas as pl
from jax.experimental.pallas import tpu as pltpu


def _se_fused_kernel(x_ref, w1_ref, b1_ref, w2_ref, b2_ref, alpha_ref, o_ref):
    x = x_ref[...]
    bt, c, hw = x.shape

    ones_col = jnp.ones((hw, 1), x.dtype)
    sums = jnp.dot(x.reshape(bt * c, hw), ones_col,
                   preferred_element_type=jnp.float32).reshape(bt, c)

    h = jnp.dot(sums, w1_ref[...],
                preferred_element_type=jnp.float32) + b1_ref[...]
    h = jnp.where(h > 0, h, alpha_ref[0] * h)
    g = jnp.dot(h, w2_ref[...],
                preferred_element_type=jnp.float32) + b2_ref[...]
    gate = jax.nn.sigmoid(g).astype(x.dtype)
    o_ref[...] = x * gate[:, :, None]


def _even_batch_tile(B, per_batch_bytes, budget_bytes):
    bt = 1
    for d in range(1, B + 1):
        if B % d == 0 and d * per_batch_bytes <= budget_bytes and B >= 2 * d:
            bt = d
    return bt


def kernel(x_nchw, w1, b1, alpha, w2, b2):
    B, C, H, W = x_nchw.shape
    HW = H * W
    Cr = w1.shape[0]
    io_dtype = x_nchw.dtype
    itemsize = jnp.dtype(io_dtype).itemsize

    x = x_nchw.reshape(B, C, HW)
    w1_t = jnp.asarray(w1, jnp.float32).T * (1.0 / HW)
    w2_t = jnp.asarray(w2, jnp.float32).T
    b1_r = jnp.asarray(b1, jnp.float32).reshape(1, Cr)
    b2_r = jnp.asarray(b2, jnp.float32).reshape(1, C)
    alpha_s = jnp.asarray(alpha, jnp.float32).reshape(1)

    per_batch = C * HW * itemsize
    Bt = _even_batch_tile(B, per_batch, budget_bytes=8 * 1024 * 1024)
    grid = (B // Bt,)

    cost = pl.CostEstimate(
        flops=3 * B * C * HW + 4 * B * C * Cr,
        transcendentals=B * C,
        bytes_accessed=2 * B * C * HW * itemsize + (2 * C * Cr + C + Cr) * 4,
    )
    out = pl.pallas_call(
        _se_fused_kernel,
        out_shape=jax.ShapeDtypeStruct((B, C, HW), io_dtype),
        grid=grid,
        in_specs=[
            pl.BlockSpec((Bt, C, HW), lambda i: (i, 0, 0)),
            pl.BlockSpec((C, Cr), lambda i: (0, 0)),
            pl.BlockSpec((1, Cr), lambda i: (0, 0)),
            pl.BlockSpec((Cr, C), lambda i: (0, 0)),
            pl.BlockSpec((1, C), lambda i: (0, 0)),
            pl.BlockSpec(memory_space=pltpu.MemorySpace.SMEM),
        ],
        out_specs=pl.BlockSpec((Bt, C, HW), lambda i: (i, 0, 0)),
        compiler_params=pltpu.CompilerParams(
            dimension_semantics=("parallel",),
            vmem_limit_bytes=56 * 1024 * 1024,
        ),
        cost_estimate=cost,
    )(x, w1_t, b1_r, w2_t, b2_r, alpha_s)
    return out.reshape(B, C, H, W)


# speedup vs baseline: 5.7310x; 5.7310x over previous
import jax
import jax.numpy as jnp
from jax.experimental import pallas as pl
from jax.experimental.pallas import tpu as pltpu


def _se_hwbc_kernel(x_ref, w1_ref, b1_ref, w2_ref, b2_ref, alpha_ref, o_ref):
    x = x_ref[...]
    hw = x.shape[0]

    chunk = 56
    partials = [
        jnp.sum(x[s:s + chunk].astype(jnp.float32), axis=0)
        for s in range(0, hw, chunk)
    ]
    sums = sum(partials[1:], partials[0])

    h = jnp.dot(sums, w1_ref[...],
                preferred_element_type=jnp.float32) + b1_ref[...]
    h = jnp.where(h > 0, h, alpha_ref[0] * h)
    g = jnp.dot(h, w2_ref[...],
                preferred_element_type=jnp.float32) + b2_ref[...]
    gate = jax.nn.sigmoid(g).astype(x.dtype)

    o_ref[...] = x * gate[None, :, :]


def _even_batch_tile(B, per_batch_bytes, budget_bytes):
    bt = 1
    for d in range(1, B + 1):
        if B % d == 0 and d * per_batch_bytes <= budget_bytes and B >= 2 * d:
            bt = d
    return bt


def kernel(x_nchw, w1, b1, alpha, w2, b2):
    B, C, H, W = x_nchw.shape
    HW = H * W
    Cr = w1.shape[0]
    io_dtype = x_nchw.dtype
    itemsize = jnp.dtype(io_dtype).itemsize

    x = jnp.transpose(x_nchw, (2, 3, 0, 1)).reshape(HW, B, C)

    w1_t = jnp.asarray(w1, jnp.float32).T * (1.0 / HW)
    w2_t = jnp.asarray(w2, jnp.float32).T
    b1_r = jnp.asarray(b1, jnp.float32).reshape(1, Cr)
    b2_r = jnp.asarray(b2, jnp.float32).reshape(1, C)
    alpha_s = jnp.asarray(alpha, jnp.float32).reshape(1)

    per_batch = HW * C * itemsize
    Bt = _even_batch_tile(B, per_batch, budget_bytes=8 * 1024 * 1024)
    grid = (B // Bt,)

    cost = pl.CostEstimate(
        flops=3 * B * C * HW + 4 * B * C * Cr,
        transcendentals=B * C,
        bytes_accessed=2 * B * C * HW * itemsize + (2 * C * Cr + C + Cr) * 4,
    )
    out = pl.pallas_call(
        _se_hwbc_kernel,
        out_shape=jax.ShapeDtypeStruct((HW, B, C), io_dtype),
        grid=grid,
        in_specs=[
            pl.BlockSpec((HW, Bt, C), lambda i: (0, i, 0)),
            pl.BlockSpec((C, Cr), lambda i: (0, 0)),
            pl.BlockSpec((1, Cr), lambda i: (0, 0)),
            pl.BlockSpec((Cr, C), lambda i: (0, 0)),
            pl.BlockSpec((1, C), lambda i: (0, 0)),
            pl.BlockSpec(memory_space=pltpu.MemorySpace.SMEM),
        ],
        out_specs=pl.BlockSpec((HW, Bt, C), lambda i: (0, i, 0)),
        compiler_params=pltpu.CompilerParams(
            dimension_semantics=("parallel",),
            vmem_limit_bytes=56 * 1024 * 1024,
        ),
        cost_estimate=cost,
    )(x, w1_t, b1_r, w2_t, b2_r, alpha_s)

    return jnp.transpose(out.reshape(H, W, B, C), (2, 3, 0, 1))


# speedup vs baseline: 5.7721x; 1.0072x over previous
import jax
import jax.numpy as jnp
from jax.experimental import pallas as pl
from jax.experimental.pallas import tpu as pltpu


def _se_hwbc_kernel(x_ref, w1_ref, b1_ref, w2_ref, b2_ref, alpha_ref, o_ref):
    x = x_ref[...]
    hw = x.shape[0]

    chunk = 56
    partials = [
        jnp.sum(x[s:s + chunk].astype(jnp.float32), axis=0)
        for s in range(0, hw, chunk)
    ]
    sums = sum(partials[1:], partials[0])

    h = jnp.dot(sums, w1_ref[...],
                preferred_element_type=jnp.float32) + b1_ref[...]
    h = jnp.where(h > 0, h, alpha_ref[0] * h)
    g = jnp.dot(h, w2_ref[...],
                preferred_element_type=jnp.float32) + b2_ref[...]
    gate = jax.nn.sigmoid(g).astype(x.dtype)

    o_ref[...] = x * gate[None, :, :]


def _even_batch_tile(B, per_batch_bytes, budget_bytes):
    bt = 1
    for d in range(1, B + 1):
        if B % d == 0 and d * per_batch_bytes <= budget_bytes and B >= 2 * d:
            bt = d
    return bt


def kernel(x_nchw, w1, b1, alpha, w2, b2):
    B, C, H, W = x_nchw.shape
    HW = H * W
    Cr = w1.shape[0]
    io_dtype = x_nchw.dtype
    itemsize = jnp.dtype(io_dtype).itemsize

    x = jnp.transpose(x_nchw, (2, 3, 0, 1)).reshape(HW, B, C)

    w1_t = jnp.asarray(w1, jnp.float32).T * (1.0 / HW)
    w2_t = jnp.asarray(w2, jnp.float32).T
    b1_r = jnp.asarray(b1, jnp.float32).reshape(1, Cr)
    b2_r = jnp.asarray(b2, jnp.float32).reshape(1, C)
    alpha_s = jnp.asarray(alpha, jnp.float32).reshape(1)

    per_batch = HW * C * itemsize
    Bt = _even_batch_tile(B, per_batch, budget_bytes=13 * 1024 * 1024)
    grid = (B // Bt,)

    cost = pl.CostEstimate(
        flops=3 * B * C * HW + 4 * B * C * Cr,
        transcendentals=B * C,
        bytes_accessed=2 * B * C * HW * itemsize + (2 * C * Cr + C + Cr) * 4,
    )
    out = pl.pallas_call(
        _se_hwbc_kernel,
        out_shape=jax.ShapeDtypeStruct((HW, B, C), io_dtype),
        grid=grid,
        in_specs=[
            pl.BlockSpec((HW, Bt, C), lambda i: (0, i, 0)),
            pl.BlockSpec((C, Cr), lambda i: (0, 0)),
            pl.BlockSpec((1, Cr), lambda i: (0, 0)),
            pl.BlockSpec((Cr, C), lambda i: (0, 0)),
            pl.BlockSpec((1, C), lambda i: (0, 0)),
            pl.BlockSpec(memory_space=pltpu.MemorySpace.SMEM),
        ],
        out_specs=pl.BlockSpec((HW, Bt, C), lambda i: (0, i, 0)),
        compiler_params=pltpu.CompilerParams(
            dimension_semantics=("parallel",),
            vmem_limit_bytes=56 * 1024 * 1024,
        ),
        cost_estimate=cost,
    )(x, w1_t, b1_r, w2_t, b2_r, alpha_s)

    return jnp.transpose(out.reshape(H, W, B, C), (2, 3, 0, 1))
